# baseline (speedup 1.0000x reference)
import functools

import jax
import jax.numpy as jnp
from jax import lax
from jax.experimental import pallas as pl
from jax.experimental.pallas import tpu as pltpu
from jax.experimental.pallas import tpu_sc as plsc

_TIME = 4096
_BATCH = 1024
_NOUT = 2
_GAP = 8
_ROWS = _TIME * 16

_RBLK = 8192
_TC_STEPS = 6
_TC_ROWS = _TC_STEPS * _RBLK
_SC_GROUPS = (_ROWS - _TC_ROWS) // 128

_N0 = float(_TIME * _BATCH)
_N1 = float((_TIME // _GAP) * _BATCH)


def _tc_kernel(o_ref, t_ref, out_ref, acc0_ref, acc1_ref):
    i = pl.program_id(0)

    o = o_ref[...]
    t = t_ref[...]

    d = o - t
    sq = d * d
    part0 = jnp.sum(sq.reshape(_RBLK // 8, 8, 128), axis=0)

    o4 = o.reshape(_RBLK // 128, 8, 16, 128)
    rowsum = jnp.sum(o4, axis=1)
    tobs = t.reshape(_RBLK // 128, 8, 16, 128)[:, 0, :, :]
    d1 = rowsum - 8.0 * tobs
    sq1 = d1 * d1
    part1 = jnp.sum(sq1, axis=0)

    @pl.when(i == 0)
    def _init():
        acc0_ref[...] = part0
        acc1_ref[...] = part1

    @pl.when(i > 0)
    def _accum():
        acc0_ref[...] += part0
        acc1_ref[...] += part1

    @pl.when(i == _TC_STEPS - 1)
    def _finish():
        row0 = jax.lax.broadcasted_iota(jnp.int32, (8, 128), 0)
        s0 = jnp.sum(jnp.where(row0 % 2 == 0, acc0_ref[...], 0.0))
        row1 = jax.lax.broadcasted_iota(jnp.int32, (16, 128), 0)
        s1 = jnp.sum(jnp.where(row1 % 2 == 1, acc1_ref[...], 0.0))
        out_ref[0, 0] = s0
        out_ref[0, 1] = s1


def _sc_kernel(o_hbm, t_hbm, out_hbm):
    def body(o_v, t_v, out_v):
        acc0s = []
        acc1s = []
        for c in range(0, 128, 16):
            a0 = None
            for m in range(0, 16, 2):
                for u in range(8):
                    sl = (pl.ds(u * 16 + m, 1), pl.ds(c, 16))
                    d = o_v[sl] - t_v[sl]
                    a0 = d * d if a0 is None else a0 + d * d
            acc0s.append(a0)
            a1 = None
            for m in range(1, 16, 2):
                s = o_v[pl.ds(m, 1), pl.ds(c, 16)]
                for u in range(1, 8):
                    s = s + o_v[pl.ds(u * 16 + m, 1), pl.ds(c, 16)]
                d1 = s - 8.0 * t_v[pl.ds(m, 1), pl.ds(c, 16)]
                a1 = d1 * d1 if a1 is None else a1 + d1 * d1
            acc1s.append(a1)
        while len(acc0s) > 1:
            acc0s = [a + b for a, b in zip(acc0s[::2], acc0s[1::2])]
        while len(acc1s) > 1:
            acc1s = [a + b for a, b in zip(acc1s[::2], acc1s[1::2])]
        out_v[pl.ds(0, 1), pl.ds(0, 16)] = acc0s[0]
        out_v[pl.ds(0, 1), pl.ds(16, 16)] = acc1s[0]
        zero = jnp.zeros((1, 16), jnp.float32)
        for c in range(32, 128, 16):
            out_v[pl.ds(0, 1), pl.ds(c, 16)] = zero

    pltpu.emit_pipeline(
        body,
        grid=(_SC_GROUPS,),
        in_specs=[
            pl.BlockSpec((128, 128), index_map=lambda g: (g + _TC_ROWS // 128, 0)),
            pl.BlockSpec((128, 128), index_map=lambda g: (g + _TC_ROWS // 128, 0)),
        ],
        out_specs=[
            pl.BlockSpec((1, 128), index_map=lambda g: (g, 0)),
        ],
        core_axis_name=("c", "s"),
        dimension_semantics=(pltpu.PARALLEL,),
    )(o_hbm, t_hbm, out_hbm)


def _rowview(x):
    return (
        x.reshape(_TIME, 8, 128, _NOUT)
        .transpose(0, 1, 3, 2)
        .reshape(_ROWS, 128)
    )


def kernel(output, target):
    o2 = _rowview(output)
    t2 = _rowview(target)

    sc_fn = functools.partial(
        pl.kernel,
        mesh=plsc.VectorSubcoreMesh(core_axis_name="c", subcore_axis_name="s"),
        out_type=jax.ShapeDtypeStruct((_SC_GROUPS, 128), jnp.float32),
    )(_sc_kernel)
    sc_part = sc_fn(o2, t2)

    tc_part = pl.pallas_call(
        _tc_kernel,
        grid=(_TC_STEPS,),
        in_specs=[
            pl.BlockSpec((_RBLK, 128), lambda i: (i, 0)),
            pl.BlockSpec((_RBLK, 128), lambda i: (i, 0)),
        ],
        out_specs=pl.BlockSpec(memory_space=pltpu.SMEM),
        out_shape=jax.ShapeDtypeStruct((1, 2), jnp.float32),
        scratch_shapes=[
            pltpu.VMEM((8, 128), jnp.float32),
            pltpu.VMEM((16, 128), jnp.float32),
        ],
    )(o2, t2)

    s0 = tc_part[0, 0] + jnp.sum(sc_part[:, 0:16])
    s1 = tc_part[0, 1] + jnp.sum(sc_part[:, 16:32])
    return 0.5 * (s0 / _N0) + 0.5 * (s1 / (64.0 * _N1))


# speedup vs baseline: 1.2592x; 1.2592x over previous
import functools

import jax
import jax.numpy as jnp
from jax import lax
from jax.experimental import pallas as pl
from jax.experimental.pallas import tpu as pltpu
from jax.experimental.pallas import tpu_sc as plsc

_TIME = 4096
_BATCH = 1024
_NOUT = 2
_GAP = 8
_ROWS = _TIME * 16

_RBLK = 8192
_TC_STEPS = 7
_TC_ROWS = _TC_STEPS * _RBLK
_SC_GROUPS = (_ROWS - _TC_ROWS) // 128

_N0 = float(_TIME * _BATCH)
_N1 = float((_TIME // _GAP) * _BATCH)


def _tc_kernel(o_ref, t_ref, out_ref, acc0_ref, acc1_ref):
    i = pl.program_id(0)

    o = o_ref[...]
    t = t_ref[...]

    d = o - t
    sq = d * d
    part0 = jnp.sum(sq.reshape(_RBLK // 8, 8, 128), axis=0)

    o4 = o.reshape(_RBLK // 128, 8, 16, 128)
    rowsum = jnp.sum(o4, axis=1)
    tobs = t.reshape(_RBLK // 128, 8, 16, 128)[:, 0, :, :]
    d1 = rowsum - 8.0 * tobs
    sq1 = d1 * d1
    part1 = jnp.sum(sq1, axis=0)

    @pl.when(i == 0)
    def _init():
        acc0_ref[...] = part0
        acc1_ref[...] = part1

    @pl.when(i > 0)
    def _accum():
        acc0_ref[...] += part0
        acc1_ref[...] += part1

    @pl.when(i == _TC_STEPS - 1)
    def _finish():
        row0 = jax.lax.broadcasted_iota(jnp.int32, (8, 128), 0)
        s0 = jnp.sum(jnp.where(row0 % 2 == 0, acc0_ref[...], 0.0))
        row1 = jax.lax.broadcasted_iota(jnp.int32, (16, 128), 0)
        s1 = jnp.sum(jnp.where(row1 % 2 == 1, acc1_ref[...], 0.0))
        out_ref[0, 0] = s0
        out_ref[0, 1] = s1


def _sc_kernel(o_hbm, t_hbm, out_hbm):
    def body(o_v, t_v, out_v):
        acc0s = []
        acc1s = []
        for c in range(0, 128, 16):
            a0 = None
            for m in range(0, 16, 2):
                for u in range(8):
                    sl = (pl.ds(u * 16 + m, 1), pl.ds(c, 16))
                    d = o_v[sl] - t_v[sl]
                    a0 = d * d if a0 is None else a0 + d * d
            acc0s.append(a0)
            a1 = None
            for m in range(1, 16, 2):
                s = o_v[pl.ds(m, 1), pl.ds(c, 16)]
                for u in range(1, 8):
                    s = s + o_v[pl.ds(u * 16 + m, 1), pl.ds(c, 16)]
                d1 = s - 8.0 * t_v[pl.ds(m, 1), pl.ds(c, 16)]
                a1 = d1 * d1 if a1 is None else a1 + d1 * d1
            acc1s.append(a1)
        while len(acc0s) > 1:
            acc0s = [a + b for a, b in zip(acc0s[::2], acc0s[1::2])]
        while len(acc1s) > 1:
            acc1s = [a + b for a, b in zip(acc1s[::2], acc1s[1::2])]
        out_v[pl.ds(0, 1), pl.ds(0, 16)] = acc0s[0]
        out_v[pl.ds(0, 1), pl.ds(16, 16)] = acc1s[0]
        zero = jnp.zeros((1, 16), jnp.float32)
        for c in range(32, 128, 16):
            out_v[pl.ds(0, 1), pl.ds(c, 16)] = zero

    pltpu.emit_pipeline(
        body,
        grid=(_SC_GROUPS,),
        in_specs=[
            pl.BlockSpec((128, 128), index_map=lambda g: (g + _TC_ROWS // 128, 0)),
            pl.BlockSpec((128, 128), index_map=lambda g: (g + _TC_ROWS // 128, 0)),
        ],
        out_specs=[
            pl.BlockSpec((1, 128), index_map=lambda g: (g, 0)),
        ],
        core_axis_name=("c", "s"),
        dimension_semantics=(pltpu.PARALLEL,),
    )(o_hbm, t_hbm, out_hbm)


def _rowview(x):
    return (
        x.reshape(_TIME, 8, 128, _NOUT)
        .transpose(0, 1, 3, 2)
        .reshape(_ROWS, 128)
    )


def kernel(output, target):
    o2 = _rowview(output)
    t2 = _rowview(target)

    sc_fn = functools.partial(
        pl.kernel,
        mesh=plsc.VectorSubcoreMesh(core_axis_name="c", subcore_axis_name="s"),
        out_type=jax.ShapeDtypeStruct((_SC_GROUPS, 128), jnp.float32),
    )(_sc_kernel)
    sc_part = sc_fn(o2, t2)

    tc_part = pl.pallas_call(
        _tc_kernel,
        grid=(_TC_STEPS,),
        in_specs=[
            pl.BlockSpec((_RBLK, 128), lambda i: (i, 0)),
            pl.BlockSpec((_RBLK, 128), lambda i: (i, 0)),
        ],
        out_specs=pl.BlockSpec(memory_space=pltpu.SMEM),
        out_shape=jax.ShapeDtypeStruct((1, 2), jnp.float32),
        scratch_shapes=[
            pltpu.VMEM((8, 128), jnp.float32),
            pltpu.VMEM((16, 128), jnp.float32),
        ],
    )(o2, t2)

    s0 = tc_part[0, 0] + jnp.sum(sc_part[:, 0:16])
    s1 = tc_part[0, 1] + jnp.sum(sc_part[:, 16:32])
    return 0.5 * (s0 / _N0) + 0.5 * (s1 / (64.0 * _N1))
